# baseline (speedup 1.0000x reference)
import functools

import jax
import jax.numpy as jnp
from jax.experimental import pallas as pl
from jax.experimental.pallas import tpu as pltpu

_SIGN = -(2**31)
_LOW31 = 0x7FFFFFFF


def _sortable(vals):
    y = jax.lax.bitcast_convert_type(vals, jnp.int32)
    return jnp.where(y >= 0, y, y ^ jnp.int32(_LOW31))


def _unsortable(keys):
    y = jnp.where(keys >= 0, keys, keys ^ jnp.int32(_LOW31))
    return jax.lax.bitcast_convert_type(y, jnp.float32)


def _bag_kernel(x_ref, pet_ref, out_ref, *, k):
    vals = x_ref[0] + pet_ref[...]
    keys = _sortable(vals)
    kf = jnp.float32(k)
    t = jnp.full((vals.shape[0], 1), jnp.int32(_SIGN), jnp.int32)
    for bit in range(31, -1, -1):
        m = jnp.int32(_SIGN) if bit == 31 else jnp.int32(1 << bit)
        cand = t ^ m
        cnt = jnp.sum((keys >= cand).astype(jnp.float32), axis=1, keepdims=True)
        t = jnp.where(cnt >= kf, cand, t)
    tv = _unsortable(t)
    gt = (vals > tv).astype(jnp.float32)
    cnt_gt = jnp.sum(gt, axis=1, keepdims=True)
    s = jnp.sum(vals * gt, axis=1, keepdims=True)
    top = s + (kf - cnt_gt) * tv
    out_ref[...] = (top / kf).T.reshape(out_ref.shape)


def _oddeven_pairs(n):
    res = []

    def merge(lo, m, r):
        step = r * 2
        if step < m:
            merge(lo, m, step)
            merge(lo + r, m, step)
            for i in range(lo + r, lo + m - r, step):
                res.append((i, i + r))
        else:
            res.append((lo, lo + r))

    def sort(lo, m):
        if m > 1:
            h = m // 2
            sort(lo, h)
            sort(lo + h, h)
            merge(lo, m, 1)

    sort(0, n)
    return res


def _cam_kernel(x_ref, pet_ref, w_ref, out_ref, cam_ref, acc_ref, *, k, ka):
    emb = x_ref[0] + pet_ref[...]
    n_clz = out_ref.shape[1]
    cam_ref[0:n_clz, :] = jnp.dot(
        w_ref[...], emb, preferred_element_type=jnp.float32)
    if cam_ref.shape[0] > n_clz:
        cam_ref[n_clz:, :] = jnp.full(
            (cam_ref.shape[0] - n_clz, cam_ref.shape[1]), -jnp.inf, jnp.float32)
    ngp = acc_ref.shape[0]
    iota_g = jax.lax.broadcasted_iota(jnp.int32, (ngp, 1), 0).astype(jnp.float32)
    iota_a = jax.lax.broadcasted_iota(jnp.int32, (ka, 1), 0).astype(jnp.float32)
    acc_ref[...] = jnp.zeros_like(acc_ref)

    nch = cam_ref.shape[0] // 16
    cam16 = cam_ref[...].reshape(nch, 16, cam_ref.shape[1])
    s = [cam16[:, j, :] for j in range(16)]
    for a_i, b_i in _oddeven_pairs(16):
        hi = jnp.maximum(s[a_i], s[b_i])
        lo = jnp.minimum(s[a_i], s[b_i])
        s[a_i], s[b_i] = hi, lo

    def body(i, carry):
        row = cam_ref[pl.ds(i, 1), :]
        b3 = s[7] > row
        b2 = jnp.where(b3, s[11], s[3]) > row
        b1 = jnp.where(b3, jnp.where(b2, s[13], s[9]),
                       jnp.where(b2, s[5], s[1])) > row
        b0 = jnp.where(b3,
                       jnp.where(b2, jnp.where(b1, s[14], s[12]),
                                 jnp.where(b1, s[10], s[8])),
                       jnp.where(b2, jnp.where(b1, s[6], s[4]),
                                 jnp.where(b1, s[2], s[0]))) > row
        ball = s[15] > row
        cch = (jnp.where(b3, 8.0, 0.0) + jnp.where(b2, 4.0, 0.0)
               + jnp.where(b1, 2.0, 0.0) + jnp.where(b0, 1.0, 0.0)
               + jnp.where(ball, 1.0, 0.0))
        cnt = jnp.sum(cch, axis=0, keepdims=True)
        fi = i.astype(jnp.float32)
        tie = jnp.sum(
            jnp.where((cam_ref[0:ka, :] == row) & (iota_a < fi), 1.0, 0.0),
            axis=0, keepdims=True)
        rank = cnt + tie
        grp = jnp.floor(rank * (1.0 / 16.0))
        low = rank - 16.0 * grp
        pw = jax.lax.bitcast_convert_type(
            (low.astype(jnp.int32) + 127) << 23, jnp.float32)
        acc_ref[...] += jnp.where(iota_g == grp, pw, 0.0)
        return carry

    jax.lax.fori_loop(0, k, body, 0)

    acc = acc_ref[...]
    bits = []
    for l in range(16):
        t = jnp.floor(acc * (0.5 ** l))
        bits.append((t - 2.0 * jnp.floor(t * 0.5))[:, None, :])
    mask = jnp.concatenate(bits, axis=1).reshape(ngp * 16, acc.shape[1])
    out_ref[0] = cam_ref[0:n_clz, :] * mask[:n_clz, :]


def _logits_kernel(bag_ref, w_ref, b_ref, out_ref):
    out_ref[...] = jax.lax.dot_general(
        bag_ref[...], w_ref[...], (((1,), (1,)), ((), ())),
        preferred_element_type=jnp.float32) + b_ref[...]


def kernel(instance_embeddings, W, b, pe):
    x = instance_embeddings
    Bb, D, N = x.shape
    n_clz = W.shape[0]
    k = int(D * 0.2)
    ka = ((k + 7) // 8) * 8
    peT = jnp.transpose(pe[0, :N, :], (1, 0))

    Dt = min(256, D)
    bag3 = pl.pallas_call(
        functools.partial(_bag_kernel, k=k),
        grid=(Bb, D // Dt),
        in_specs=[
            pl.BlockSpec((1, Dt, N), lambda bb, dt: (bb, dt, 0)),
            pl.BlockSpec((Dt, N), lambda bb, dt: (dt, 0)),
        ],
        out_specs=pl.BlockSpec((1, 1, Dt), lambda bb, dt: (bb, 0, dt)),
        out_shape=jax.ShapeDtypeStruct((Bb, 1, D), jnp.float32),
    )(x, peT)
    bag = bag3.reshape(Bb, D)

    Nt = min(512, N)
    ngp = ((((n_clz + 15) // 16) + 7) // 8) * 8
    rap_cam = pl.pallas_call(
        functools.partial(_cam_kernel, k=k, ka=ka),
        grid=(Bb, N // Nt),
        in_specs=[
            pl.BlockSpec((1, D, Nt), lambda bb, j: (bb, 0, j)),
            pl.BlockSpec((D, Nt), lambda bb, j: (0, j)),
            pl.BlockSpec((n_clz, D), lambda bb, j: (0, 0)),
        ],
        out_specs=pl.BlockSpec((1, n_clz, Nt), lambda bb, j: (bb, 0, j)),
        out_shape=jax.ShapeDtypeStruct((Bb, n_clz, N), jnp.float32),
        scratch_shapes=[pltpu.VMEM((((n_clz + 15) // 16) * 16, Nt), jnp.float32),
                        pltpu.VMEM((ngp, Nt), jnp.float32)],
    )(x, peT, W)

    bag_logits = pl.pallas_call(
        _logits_kernel,
        in_specs=[
            pl.BlockSpec((Bb, D), lambda: (0, 0)),
            pl.BlockSpec((n_clz, D), lambda: (0, 0)),
            pl.BlockSpec((1, n_clz), lambda: (0, 0)),
        ],
        out_specs=pl.BlockSpec((Bb, n_clz), lambda: (0, 0)),
        out_shape=jax.ShapeDtypeStruct((Bb, n_clz), jnp.float32),
    )(bag, W, b.reshape(1, n_clz))

    return (bag_logits, rap_cam, bag, x)


# speedup vs baseline: 7.3507x; 7.3507x over previous
import functools

import jax
import jax.numpy as jnp
from jax.experimental import pallas as pl
from jax.experimental.pallas import tpu as pltpu

_SIGN = -(2**31)
_LOW31 = 0x7FFFFFFF


def _sortable(vals):
    y = jax.lax.bitcast_convert_type(vals, jnp.int32)
    return jnp.where(y >= 0, y, y ^ jnp.int32(_LOW31))


def _unsortable(keys):
    y = jnp.where(keys >= 0, keys, keys ^ jnp.int32(_LOW31))
    return jax.lax.bitcast_convert_type(y, jnp.float32)


def _bag_kernel(x_ref, pet_ref, out_ref, *, k):
    vals = x_ref[0] + pet_ref[...]
    keys = _sortable(vals)
    kf = jnp.float32(k)
    t = jnp.full((vals.shape[0], 1), jnp.int32(_SIGN), jnp.int32)
    for bit in range(31, -1, -1):
        m = jnp.int32(_SIGN) if bit == 31 else jnp.int32(1 << bit)
        cand = t ^ m
        cnt = jnp.sum((keys >= cand).astype(jnp.float32), axis=1, keepdims=True)
        t = jnp.where(cnt >= kf, cand, t)
    tv = _unsortable(t)
    gt = (vals > tv).astype(jnp.float32)
    cnt_gt = jnp.sum(gt, axis=1, keepdims=True)
    s = jnp.sum(vals * gt, axis=1, keepdims=True)
    top = s + (kf - cnt_gt) * tv
    out_ref[...] = (top / kf).T.reshape(out_ref.shape)


def _nr_of(n_clz):
    nch = (((n_clz + 15) // 16) * 16) // 16
    return ((nch + 7) // 8) * 8


def _oddeven_pairs(n):
    res = []

    def merge(lo, m, r):
        step = r * 2
        if step < m:
            merge(lo, m, step)
            merge(lo + r, m, step)
            for i in range(lo + r, lo + m - r, step):
                res.append((i, i + r))
        else:
            res.append((lo, lo + r))

    def sort(lo, m):
        if m > 1:
            h = m // 2
            sort(lo, h)
            sort(lo + h, h)
            merge(lo, m, 1)

    sort(0, n)
    return res


def _cam_kernel(x_ref, pet_ref, w_ref, out_ref, cam_ref, acc_ref, srt_ref, *, k, ka):
    emb = x_ref[0] + pet_ref[...]
    n_clz = out_ref.shape[1]
    cam_ref[0:n_clz, :] = jnp.dot(
        w_ref[...], emb, preferred_element_type=jnp.float32)
    if cam_ref.shape[0] > n_clz:
        cam_ref[n_clz:, :] = jnp.full(
            (cam_ref.shape[0] - n_clz, cam_ref.shape[1]), -jnp.inf, jnp.float32)
    ngp = acc_ref.shape[0]
    iota_g = jax.lax.broadcasted_iota(jnp.int32, (ngp, 1), 0).astype(jnp.float32)
    iota_a = jax.lax.broadcasted_iota(jnp.int32, (ka, 1), 0).astype(jnp.float32)
    acc_ref[...] = jnp.zeros_like(acc_ref)

    nch = cam_ref.shape[0] // 16
    nt = cam_ref.shape[1]
    nr = ((nch + 7) // 8) * 8
    cam16 = cam_ref[...].reshape(nch, 16, nt)
    s = [cam16[:, j, :] for j in range(16)]
    for a_i, b_i in _oddeven_pairs(16):
        hi = jnp.maximum(s[a_i], s[b_i])
        lo = jnp.minimum(s[a_i], s[b_i])
        s[a_i], s[b_i] = hi, lo
    pad = jnp.full((nr - nch, nt), -jnp.inf, jnp.float32)
    for j in range(16):
        srt_ref[j * nr:(j + 1) * nr, :] = jnp.concatenate([s[j], pad], axis=0)

    def sj(j):
        return srt_ref[j * nr:(j + 1) * nr, :]

    def body(i, carry):
        row = cam_ref[pl.ds(i, 1), :]
        b3 = sj(7) > row
        b2 = jnp.where(b3, sj(11), sj(3)) > row
        b1 = jnp.where(b3, jnp.where(b2, sj(13), sj(9)),
                       jnp.where(b2, sj(5), sj(1))) > row
        b0 = jnp.where(b3,
                       jnp.where(b2, jnp.where(b1, sj(14), sj(12)),
                                 jnp.where(b1, sj(10), sj(8))),
                       jnp.where(b2, jnp.where(b1, sj(6), sj(4)),
                                 jnp.where(b1, sj(2), sj(0)))) > row
        ball = sj(15) > row
        cch = (jnp.where(b3, 8.0, 0.0) + jnp.where(b2, 4.0, 0.0)
               + jnp.where(b1, 2.0, 0.0) + jnp.where(b0, 1.0, 0.0)
               + jnp.where(ball, 1.0, 0.0))
        cnt = jnp.sum(cch, axis=0, keepdims=True)
        fi = i.astype(jnp.float32)
        tie = jnp.sum(
            jnp.where((cam_ref[0:ka, :] == row) & (iota_a < fi), 1.0, 0.0),
            axis=0, keepdims=True)
        rank = cnt + tie
        grp = jnp.floor(rank * (1.0 / 16.0))
        low = rank - 16.0 * grp
        pw = jax.lax.bitcast_convert_type(
            (low.astype(jnp.int32) + 127) << 23, jnp.float32)
        acc_ref[...] += jnp.where(iota_g == grp, pw, 0.0)
        return carry

    jax.lax.fori_loop(0, k, body, 0)

    acc = acc_ref[...]
    bits = []
    for l in range(16):
        t = jnp.floor(acc * (0.5 ** l))
        bits.append((t - 2.0 * jnp.floor(t * 0.5))[:, None, :])
    mask = jnp.concatenate(bits, axis=1).reshape(ngp * 16, acc.shape[1])
    out_ref[0] = cam_ref[0:n_clz, :] * mask[:n_clz, :]


def _logits_kernel(bag_ref, w_ref, b_ref, out_ref):
    out_ref[...] = jax.lax.dot_general(
        bag_ref[...], w_ref[...], (((1,), (1,)), ((), ())),
        preferred_element_type=jnp.float32) + b_ref[...]


def kernel(instance_embeddings, W, b, pe):
    x = instance_embeddings
    Bb, D, N = x.shape
    n_clz = W.shape[0]
    k = int(D * 0.2)
    ka = ((k + 7) // 8) * 8
    peT = jnp.transpose(pe[0, :N, :], (1, 0))

    Dt = min(256, D)
    bag3 = pl.pallas_call(
        functools.partial(_bag_kernel, k=k),
        grid=(Bb, D // Dt),
        in_specs=[
            pl.BlockSpec((1, Dt, N), lambda bb, dt: (bb, dt, 0)),
            pl.BlockSpec((Dt, N), lambda bb, dt: (dt, 0)),
        ],
        out_specs=pl.BlockSpec((1, 1, Dt), lambda bb, dt: (bb, 0, dt)),
        out_shape=jax.ShapeDtypeStruct((Bb, 1, D), jnp.float32),
    )(x, peT)
    bag = bag3.reshape(Bb, D)

    Nt = min(512, N)
    ngp = ((((n_clz + 15) // 16) + 7) // 8) * 8
    rap_cam = pl.pallas_call(
        functools.partial(_cam_kernel, k=k, ka=ka),
        grid=(Bb, N // Nt),
        in_specs=[
            pl.BlockSpec((1, D, Nt), lambda bb, j: (bb, 0, j)),
            pl.BlockSpec((D, Nt), lambda bb, j: (0, j)),
            pl.BlockSpec((n_clz, D), lambda bb, j: (0, 0)),
        ],
        out_specs=pl.BlockSpec((1, n_clz, Nt), lambda bb, j: (bb, 0, j)),
        out_shape=jax.ShapeDtypeStruct((Bb, n_clz, N), jnp.float32),
        scratch_shapes=[pltpu.VMEM((((n_clz + 15) // 16) * 16, Nt), jnp.float32),
                        pltpu.VMEM((ngp, Nt), jnp.float32),
                        pltpu.VMEM((16 * _nr_of(n_clz), Nt), jnp.float32)],
    )(x, peT, W)

    bag_logits = pl.pallas_call(
        _logits_kernel,
        in_specs=[
            pl.BlockSpec((Bb, D), lambda: (0, 0)),
            pl.BlockSpec((n_clz, D), lambda: (0, 0)),
            pl.BlockSpec((1, n_clz), lambda: (0, 0)),
        ],
        out_specs=pl.BlockSpec((Bb, n_clz), lambda: (0, 0)),
        out_shape=jax.ShapeDtypeStruct((Bb, n_clz), jnp.float32),
    )(bag, W, b.reshape(1, n_clz))

    return (bag_logits, rap_cam, bag, x)


# speedup vs baseline: 9.6716x; 1.3157x over previous
import functools

import jax
import jax.numpy as jnp
from jax.experimental import pallas as pl
from jax.experimental.pallas import tpu as pltpu

_SIGN = -(2**31)
_LOW31 = 0x7FFFFFFF


def _sortable(vals):
    y = jax.lax.bitcast_convert_type(vals, jnp.int32)
    return jnp.where(y >= 0, y, y ^ jnp.int32(_LOW31))


def _unsortable(keys):
    y = jnp.where(keys >= 0, keys, keys ^ jnp.int32(_LOW31))
    return jax.lax.bitcast_convert_type(y, jnp.float32)


def _bag_kernel(x_ref, pet_ref, out_ref, *, k):
    vals = x_ref[0] + pet_ref[...]
    keys = _sortable(vals)
    kf = jnp.float32(k)
    t = jnp.full((vals.shape[0], 1), jnp.int32(_SIGN), jnp.int32)
    for bit in range(31, -1, -1):
        m = jnp.int32(_SIGN) if bit == 31 else jnp.int32(1 << bit)
        cand = t ^ m
        cnt = jnp.sum((keys >= cand).astype(jnp.float32), axis=1, keepdims=True)
        t = jnp.where(cnt >= kf, cand, t)
    tv = _unsortable(t)
    gt = (vals > tv).astype(jnp.float32)
    cnt_gt = jnp.sum(gt, axis=1, keepdims=True)
    s = jnp.sum(vals * gt, axis=1, keepdims=True)
    top = s + (kf - cnt_gt) * tv
    out_ref[...] = (top / kf).T.reshape(out_ref.shape)


def _cam_kernel(x_ref, pet_ref, w_ref, out_ref, cam_ref, acc_ref, *, k, ka):
    emb = x_ref[0] + pet_ref[...]
    n_clz = out_ref.shape[1]
    cam_ref[0:n_clz, :] = jnp.dot(
        w_ref[...], emb, preferred_element_type=jnp.float32)
    if cam_ref.shape[0] > n_clz:
        cam_ref[n_clz:, :] = jnp.full(
            (cam_ref.shape[0] - n_clz, cam_ref.shape[1]), -jnp.inf, jnp.float32)
    ngp = acc_ref.shape[0]
    iota_g = jax.lax.broadcasted_iota(jnp.int32, (ngp, 1), 0).astype(jnp.float32)
    acc_ref[...] = jnp.zeros_like(acc_ref)

    def body(i, carry):
        row = cam_ref[pl.ds(i, 1), :]
        gt = (cam_ref[...] > row).astype(jnp.float32)
        rank = jnp.sum(gt, axis=0, keepdims=True)
        grp = jnp.floor(rank * (1.0 / 16.0))
        low = rank - 16.0 * grp
        pw = jax.lax.bitcast_convert_type(
            (low.astype(jnp.int32) + 127) << 23, jnp.float32)
        acc_ref[...] += jnp.where(iota_g == grp, pw, 0.0)
        return carry

    jax.lax.fori_loop(0, k, body, 0)

    acc = acc_ref[...]
    bits = []
    for l in range(16):
        t = jnp.floor(acc * (0.5 ** l))
        bits.append((t - 2.0 * jnp.floor(t * 0.5))[:, None, :])
    mask = jnp.concatenate(bits, axis=1).reshape(ngp * 16, acc.shape[1])
    out_ref[0] = cam_ref[0:n_clz, :] * mask[:n_clz, :]


def _logits_kernel(bag_ref, w_ref, b_ref, out_ref):
    out_ref[...] = jax.lax.dot_general(
        bag_ref[...], w_ref[...], (((1,), (1,)), ((), ())),
        preferred_element_type=jnp.float32) + b_ref[...]


def kernel(instance_embeddings, W, b, pe):
    x = instance_embeddings
    Bb, D, N = x.shape
    n_clz = W.shape[0]
    k = int(D * 0.2)
    ka = ((k + 7) // 8) * 8
    peT = jnp.transpose(pe[0, :N, :], (1, 0))

    Dt = min(256, D)
    bag3 = pl.pallas_call(
        functools.partial(_bag_kernel, k=k),
        grid=(Bb, D // Dt),
        in_specs=[
            pl.BlockSpec((1, Dt, N), lambda bb, dt: (bb, dt, 0)),
            pl.BlockSpec((Dt, N), lambda bb, dt: (dt, 0)),
        ],
        out_specs=pl.BlockSpec((1, 1, Dt), lambda bb, dt: (bb, 0, dt)),
        out_shape=jax.ShapeDtypeStruct((Bb, 1, D), jnp.float32),
    )(x, peT)
    bag = bag3.reshape(Bb, D)

    Nt = min(512, N)
    ngp = ((((n_clz + 15) // 16) + 7) // 8) * 8
    rap_cam = pl.pallas_call(
        functools.partial(_cam_kernel, k=k, ka=ka),
        grid=(Bb, N // Nt),
        in_specs=[
            pl.BlockSpec((1, D, Nt), lambda bb, j: (bb, 0, j)),
            pl.BlockSpec((D, Nt), lambda bb, j: (0, j)),
            pl.BlockSpec((n_clz, D), lambda bb, j: (0, 0)),
        ],
        out_specs=pl.BlockSpec((1, n_clz, Nt), lambda bb, j: (bb, 0, j)),
        out_shape=jax.ShapeDtypeStruct((Bb, n_clz, N), jnp.float32),
        scratch_shapes=[pltpu.VMEM((((n_clz + 15) // 16) * 16, Nt), jnp.float32),
                        pltpu.VMEM((ngp, Nt), jnp.float32)],
    )(x, peT, W)

    bag_logits = pl.pallas_call(
        _logits_kernel,
        in_specs=[
            pl.BlockSpec((Bb, D), lambda: (0, 0)),
            pl.BlockSpec((n_clz, D), lambda: (0, 0)),
            pl.BlockSpec((1, n_clz), lambda: (0, 0)),
        ],
        out_specs=pl.BlockSpec((Bb, n_clz), lambda: (0, 0)),
        out_shape=jax.ShapeDtypeStruct((Bb, n_clz), jnp.float32),
    )(bag, W, b.reshape(1, n_clz))

    return (bag_logits, rap_cam, bag, x)


# speedup vs baseline: 10.8746x; 1.1244x over previous
import functools

import jax
import jax.numpy as jnp
from jax.experimental import pallas as pl
from jax.experimental.pallas import tpu as pltpu

_SIGN = -(2**31)
_LOW31 = 0x7FFFFFFF


def _sortable(vals):
    y = jax.lax.bitcast_convert_type(vals, jnp.int32)
    return jnp.where(y >= 0, y, y ^ jnp.int32(_LOW31))


def _unsortable(keys):
    y = jnp.where(keys >= 0, keys, keys ^ jnp.int32(_LOW31))
    return jax.lax.bitcast_convert_type(y, jnp.float32)


def _bag_kernel(x_ref, pet_ref, out_ref, *, k):
    vals = x_ref[0] + pet_ref[...]
    keys = _sortable(vals)
    kf = jnp.float32(k)
    t = jnp.full((vals.shape[0], 1), jnp.int32(_SIGN), jnp.int32)
    for bit in range(31, -1, -1):
        m = jnp.int32(_SIGN) if bit == 31 else jnp.int32(1 << bit)
        cand = t ^ m
        cnt = jnp.sum((keys >= cand).astype(jnp.float32), axis=1, keepdims=True)
        t = jnp.where(cnt >= kf, cand, t)
    tv = _unsortable(t)
    gt = (vals > tv).astype(jnp.float32)
    cnt_gt = jnp.sum(gt, axis=1, keepdims=True)
    s = jnp.sum(vals * gt, axis=1, keepdims=True)
    top = s + (kf - cnt_gt) * tv
    out_ref[...] = (top / kf).T.reshape(out_ref.shape)


def _cam_kernel(x_ref, pet_ref, w_ref, out_ref, cam_ref, acc_ref, *, k, ka):
    emb = x_ref[0] + pet_ref[...]
    n_clz = out_ref.shape[1]
    cam_ref[0:n_clz, :] = jnp.dot(
        w_ref[...], emb, preferred_element_type=jnp.float32)
    if cam_ref.shape[0] > n_clz:
        cam_ref[n_clz:, :] = jnp.full(
            (cam_ref.shape[0] - n_clz, cam_ref.shape[1]), -jnp.inf, jnp.float32)
    ngp = acc_ref.shape[0]
    iota_g = jax.lax.broadcasted_iota(jnp.int32, (ngp, 1), 0).astype(jnp.float32)
    acc_ref[...] = jnp.zeros_like(acc_ref)

    def body(i, carry):
        row = cam_ref[pl.ds(i, 1), :]
        gt = (cam_ref[...] > row).astype(jnp.float32)
        rank = jnp.sum(gt, axis=0, keepdims=True)
        grp = jnp.floor(rank * (1.0 / 16.0))
        low = rank - 16.0 * grp
        pw = jax.lax.bitcast_convert_type(
            (low.astype(jnp.int32) + 127) << 23, jnp.float32)
        acc_ref[...] += jnp.where(iota_g == grp, pw, 0.0)
        return carry

    jax.lax.fori_loop(0, k, body, 0)

    acc = acc_ref[...]
    bits = []
    for l in range(16):
        t = jnp.floor(acc * (0.5 ** l))
        bits.append((t - 2.0 * jnp.floor(t * 0.5))[:, None, :])
    mask = jnp.concatenate(bits, axis=1).reshape(ngp * 16, acc.shape[1])
    out_ref[0] = cam_ref[0:n_clz, :] * mask[:n_clz, :]


def _logits_kernel(bag_ref, w_ref, b_ref, out_ref):
    out_ref[...] = jax.lax.dot_general(
        bag_ref[...], w_ref[...], (((1,), (1,)), ((), ())),
        preferred_element_type=jnp.float32) + b_ref[...]


def kernel(instance_embeddings, W, b, pe):
    x = instance_embeddings
    Bb, D, N = x.shape
    n_clz = W.shape[0]
    k = int(D * 0.2)
    ka = ((k + 7) // 8) * 8
    peT = jnp.transpose(pe[0, :N, :], (1, 0))

    Dt = min(256, D)
    bag3 = pl.pallas_call(
        functools.partial(_bag_kernel, k=k),
        grid=(Bb, D // Dt),
        in_specs=[
            pl.BlockSpec((1, Dt, N), lambda bb, dt: (bb, dt, 0)),
            pl.BlockSpec((Dt, N), lambda bb, dt: (dt, 0)),
        ],
        out_specs=pl.BlockSpec((1, 1, Dt), lambda bb, dt: (bb, 0, dt)),
        out_shape=jax.ShapeDtypeStruct((Bb, 1, D), jnp.float32),
    )(x, peT)
    bag = bag3.reshape(Bb, D)

    Nt = min(1024, N)
    ngp = ((((n_clz + 15) // 16) + 7) // 8) * 8
    rap_cam = pl.pallas_call(
        functools.partial(_cam_kernel, k=k, ka=ka),
        grid=(Bb, N // Nt),
        in_specs=[
            pl.BlockSpec((1, D, Nt), lambda bb, j: (bb, 0, j)),
            pl.BlockSpec((D, Nt), lambda bb, j: (0, j)),
            pl.BlockSpec((n_clz, D), lambda bb, j: (0, 0)),
        ],
        out_specs=pl.BlockSpec((1, n_clz, Nt), lambda bb, j: (bb, 0, j)),
        out_shape=jax.ShapeDtypeStruct((Bb, n_clz, N), jnp.float32),
        scratch_shapes=[pltpu.VMEM((((n_clz + 15) // 16) * 16, Nt), jnp.float32),
                        pltpu.VMEM((ngp, Nt), jnp.float32)],
    )(x, peT, W)

    bag_logits = pl.pallas_call(
        _logits_kernel,
        in_specs=[
            pl.BlockSpec((Bb, D), lambda: (0, 0)),
            pl.BlockSpec((n_clz, D), lambda: (0, 0)),
            pl.BlockSpec((1, n_clz), lambda: (0, 0)),
        ],
        out_specs=pl.BlockSpec((Bb, n_clz), lambda: (0, 0)),
        out_shape=jax.ShapeDtypeStruct((Bb, n_clz), jnp.float32),
    )(bag, W, b.reshape(1, n_clz))

    return (bag_logits, rap_cam, bag, x)
